# baseline (speedup 1.0000x reference)
import functools
import math

import jax
import jax.numpy as jnp
from jax import lax
from jax.experimental import pallas as pl
from jax.experimental.pallas import tpu as pltpu
from jax.experimental.pallas import tpu_sc as plsc

D_MODEL = 64
SCALE = math.sqrt(D_MODEL)

NUM_CORES = 2
NUM_SUBCORES = 16
NUM_WORKERS = NUM_CORES * NUM_SUBCORES

CHUNK = 128
LANES = 16


def _make_kernel(n_tok: int, vocab: int, d: int):
  assert n_tok % (NUM_WORKERS * CHUNK) == 0
  tok_per_w = n_tok // NUM_WORKERS
  n_chunks = tok_per_w // CHUNK

  mesh = plsc.VectorSubcoreMesh(core_axis_name="c", subcore_axis_name="s")

  @functools.partial(
      pl.kernel,
      mesh=mesh,
      out_type=jax.ShapeDtypeStruct((n_tok, d), jnp.float32),
      scratch_types=[
          pltpu.VMEM((n_chunks, CHUNK), jnp.int32),
          pltpu.VMEM((CHUNK, d), jnp.float32),
          pltpu.SemaphoreType.DMA,
      ],
      compiler_params=pltpu.CompilerParams(use_tc_tiling_on_sc=False),
  )
  def emb_kernel(x_hbm, w_hbm, out_hbm, idx_v, rows_v, sem):
    wid = lax.axis_index("s") * NUM_CORES + lax.axis_index("c")
    base = wid * tok_per_w
    pltpu.sync_copy(x_hbm.at[wid], idx_v)

    def chunk_body(j, carry):
      pltpu.async_copy(w_hbm.at[idx_v.at[j]], rows_v, sem).wait()

      def row_body(r, c2):
        for col in range(d // LANES):
          sl = pl.ds(col * LANES, LANES)
          rows_v[r, sl] = rows_v[r, sl] * SCALE
        return c2

      lax.fori_loop(0, CHUNK, row_body, 0, unroll=4)
      pltpu.sync_copy(rows_v, out_hbm.at[pl.ds(base + j * CHUNK, CHUNK)])
      return carry

    lax.fori_loop(0, n_chunks, chunk_body, 0)

  return emb_kernel


def kernel(x, weight):
  b, s = x.shape
  vocab, d = weight.shape
  n_tok = b * s
  xf = x.reshape(NUM_WORKERS, n_tok // (NUM_WORKERS * CHUNK), CHUNK)
  xf = xf.astype(jnp.int32)
  out = _make_kernel(n_tok, vocab, d)(xf, weight)
  return out.reshape(b, s, d)


# speedup vs baseline: 1.1493x; 1.1493x over previous
import functools
import math

import jax
import jax.numpy as jnp
from jax import lax
from jax.experimental import pallas as pl
from jax.experimental.pallas import tpu as pltpu
from jax.experimental.pallas import tpu_sc as plsc

D_MODEL = 64
SCALE = math.sqrt(D_MODEL)

NUM_CORES = 2
NUM_SUBCORES = 16
NUM_WORKERS = NUM_CORES * NUM_SUBCORES

CHUNK = 128
LANES = 16


NBUF = 4


def _make_kernel(n_tok: int, vocab: int, d: int):
  assert n_tok % (NUM_WORKERS * CHUNK * NBUF) == 0
  tok_per_w = n_tok // NUM_WORKERS
  n_chunks = tok_per_w // CHUNK
  n_groups = n_chunks // NBUF

  mesh = plsc.VectorSubcoreMesh(core_axis_name="c", subcore_axis_name="s")

  @functools.partial(
      pl.kernel,
      mesh=mesh,
      out_type=jax.ShapeDtypeStruct((n_tok, d), jnp.float32),
      scratch_types=[
          pltpu.VMEM((n_chunks, CHUNK), jnp.int32),
          [pltpu.VMEM((CHUNK, d), jnp.float32) for _ in range(NBUF)],
          [pltpu.SemaphoreType.DMA for _ in range(NBUF)],
          [pltpu.SemaphoreType.DMA for _ in range(NBUF)],
      ],
      compiler_params=pltpu.CompilerParams(use_tc_tiling_on_sc=False),
  )
  def emb_kernel(x_hbm, w_hbm, out_hbm, idx_v, rows, gat_sem, scat_sem):
    wid = lax.axis_index("s") * NUM_CORES + lax.axis_index("c")
    base = wid * tok_per_w
    pltpu.sync_copy(x_hbm.at[wid], idx_v)

    def gather_start(j, b):
      pltpu.make_async_copy(w_hbm.at[idx_v.at[j]], rows[b], gat_sem[b]).start()

    def gather_wait(j, b):
      pltpu.make_async_copy(w_hbm.at[idx_v.at[j]], rows[b], gat_sem[b]).wait()

    def scat_start(j, b):
      dst = out_hbm.at[pl.ds(base + j * CHUNK, CHUNK)]
      pltpu.make_async_copy(rows[b], dst, scat_sem[b]).start()

    def scat_wait(j, b):
      dst = out_hbm.at[pl.ds(base + j * CHUNK, CHUNK)]
      pltpu.make_async_copy(rows[b], dst, scat_sem[b]).wait()

    def scale(b):
      def row_body(r, c2):
        for col in range(d // LANES):
          sl = pl.ds(col * LANES, LANES)
          rows[b][r, sl] = rows[b][r, sl] * SCALE
        return c2

      lax.fori_loop(0, CHUNK, row_body, 0, unroll=4)

    for b in range(NBUF):
      gather_start(b, b)

    def group_body(g, carry):
      j0 = g * NBUF
      for b in range(NBUF):
        gather_wait(j0 + b, b)
        scale(b)
        scat_start(j0 + b, b)
      for b in range(NBUF):
        scat_wait(j0 + b, b)
        gather_start(j0 + NBUF + b, b)
      return carry

    lax.fori_loop(0, n_groups - 1, group_body, 0)

    j0 = (n_groups - 1) * NBUF
    for b in range(NBUF):
      gather_wait(j0 + b, b)
      scale(b)
      scat_start(j0 + b, b)
    for b in range(NBUF):
      scat_wait(j0 + b, b)

  return emb_kernel


def kernel(x, weight):
  b, s = x.shape
  vocab, d = weight.shape
  n_tok = b * s
  xf = x.reshape(NUM_WORKERS, n_tok // (NUM_WORKERS * CHUNK), CHUNK)
  xf = xf.astype(jnp.int32)
  out = _make_kernel(n_tok, vocab, d)(xf, weight)
  return out.reshape(b, s, d)


# speedup vs baseline: 1.1584x; 1.0079x over previous
import functools
import math

import jax
import jax.numpy as jnp
from jax import lax
from jax.experimental import pallas as pl
from jax.experimental.pallas import tpu as pltpu
from jax.experimental.pallas import tpu_sc as plsc

D_MODEL = 64
SCALE = math.sqrt(D_MODEL)

NUM_CORES = 2
NUM_SUBCORES = 16
NUM_WORKERS = NUM_CORES * NUM_SUBCORES

LANES = 16
SPLIT = 104
NBUF = 4


def _make_kernel(b: int, s: int, vocab: int, d: int):
  assert b % (NUM_WORKERS * NBUF) == 0 and s % 8 == 0 and SPLIT % 8 == 0
  rows_per_w = b // NUM_WORKERS
  n_groups = rows_per_w // NBUF

  mesh = plsc.VectorSubcoreMesh(core_axis_name="c", subcore_axis_name="s")

  @functools.partial(
      pl.kernel,
      mesh=mesh,
      out_type=jax.ShapeDtypeStruct((b, s, d), jnp.float32),
      scratch_types=[
          pltpu.VMEM((rows_per_w, s), jnp.int32),
          [pltpu.VMEM((s, d), jnp.float32) for _ in range(NBUF)],
          [pltpu.SemaphoreType.DMA for _ in range(NBUF)],
          [pltpu.SemaphoreType.DMA for _ in range(NBUF)],
      ],
      compiler_params=pltpu.CompilerParams(use_tc_tiling_on_sc=False),
  )
  def emb_kernel(x_hbm, w_hbm, out_hbm, idx_v, rows, gat_sem, scat_sem):
    wid = lax.axis_index("s") * NUM_CORES + lax.axis_index("c")
    base = wid * rows_per_w
    pltpu.sync_copy(x_hbm.at[wid], idx_v)

    def gather_start(r, buf):
      pltpu.make_async_copy(
          w_hbm.at[idx_v.at[r, pl.ds(0, SPLIT)]],
          rows[buf].at[pl.ds(0, SPLIT)],
          gat_sem[buf],
      ).start()
      pltpu.make_async_copy(
          w_hbm.at[idx_v.at[r, pl.ds(SPLIT, s - SPLIT)]],
          rows[buf].at[pl.ds(SPLIT, s - SPLIT)],
          gat_sem[buf],
      ).start()

    def gather_wait(r, buf):
      pltpu.make_async_copy(
          w_hbm.at[idx_v.at[r, pl.ds(0, SPLIT)]],
          rows[buf].at[pl.ds(0, SPLIT)],
          gat_sem[buf],
      ).wait()
      pltpu.make_async_copy(
          w_hbm.at[idx_v.at[r, pl.ds(SPLIT, s - SPLIT)]],
          rows[buf].at[pl.ds(SPLIT, s - SPLIT)],
          gat_sem[buf],
      ).wait()

    def scat_start(r, buf):
      pltpu.make_async_copy(rows[buf], out_hbm.at[base + r], scat_sem[buf]).start()

    def scat_wait(r, buf):
      pltpu.make_async_copy(rows[buf], out_hbm.at[base + r], scat_sem[buf]).wait()

    def scale(buf):
      def tok_body(t, c2):
        for col in range(d // LANES):
          sl = pl.ds(col * LANES, LANES)
          rows[buf][t, sl] = rows[buf][t, sl] * SCALE
        return c2

      lax.fori_loop(0, s, tok_body, 0, unroll=4)

    for buf in range(NBUF):
      gather_start(buf, buf)

    def group_body(g, carry):
      r0 = g * NBUF
      for buf in range(NBUF):
        gather_wait(r0 + buf, buf)
        scale(buf)
        scat_start(r0 + buf, buf)
      for buf in range(NBUF):
        scat_wait(r0 + buf, buf)
        gather_start(r0 + NBUF + buf, buf)
      return carry

    lax.fori_loop(0, n_groups - 1, group_body, 0)

    r0 = (n_groups - 1) * NBUF
    for buf in range(NBUF):
      gather_wait(r0 + buf, buf)
      scale(buf)
      scat_start(r0 + buf, buf)
    for buf in range(NBUF):
      scat_wait(r0 + buf, buf)

  return emb_kernel


def kernel(x, weight):
  b, s = x.shape
  vocab, d = weight.shape
  x3 = x.reshape(NUM_WORKERS, b // NUM_WORKERS, s).astype(jnp.int32)
  return _make_kernel(b, s, vocab, d)(x3, weight)
